# baseline (speedup 1.0000x reference)
import functools

import jax
import jax.numpy as jnp
from jax import lax
from jax.experimental import pallas as pl
from jax.experimental.pallas import tpu as pltpu
from jax.experimental.pallas import tpu_sc as plsc

NC = 2
NS = 16
L = 16
NW = NC * NS
CHUNK = 128
BLK = 16
RBLK = 8


def _mesh():
    return plsc.VectorSubcoreMesh(
        core_axis_name="c", subcore_axis_name="s", num_cores=NC, num_subcores=NS
    )


@functools.lru_cache(maxsize=None)
def _make_degree(n_pad, slabs):
    rpt = n_pad // NS

    @functools.partial(
        pl.kernel,
        out_type=jax.ShapeDtypeStruct((NC, n_pad), jnp.float32),
        mesh=_mesh(),
        scratch_types=[
            pltpu.VMEM((BLK, CHUNK), jnp.int32),
            pltpu.VMEM((BLK, CHUNK), jnp.float32),
            pltpu.VMEM_SHARED((n_pad,), jnp.float32),
            pltpu.SemaphoreType.DMA,
        ],
    )
    def deg_kernel(col_h, attr_h, zero_h, out_h, colb, attrb, acc, ssem):
        c = lax.axis_index("c")
        s = lax.axis_index("s")
        pltpu.sync_copy(zero_h, acc.at[pl.ds(s * rpt, rpt)])
        plsc.subcore_barrier()
        base = (c * NS + s) * slabs * BLK

        def slab(i, carry):
            st = base + i * BLK
            pltpu.sync_copy(col_h.at[pl.ds(st, BLK)], colb)
            pltpu.sync_copy(attr_h.at[pl.ds(st, BLK)], attrb)
            descs = [
                pltpu.async_copy(attrb.at[j], acc.at[colb.at[j]], ssem, add=True)
                for j in range(BLK)
            ]
            for d in descs:
                d.wait()
            return carry

        lax.fori_loop(0, slabs, slab, 0)
        plsc.subcore_barrier()
        pltpu.sync_copy(acc.at[pl.ds(s * rpt, rpt)], out_h.at[c, pl.ds(s * rpt, rpt)])

    return deg_kernel


@functools.lru_cache(maxsize=None)
def _make_hop_planar(n_pad, slabs, combine):
    rpt = n_pad // NS

    scratch = [
        pltpu.VMEM((BLK, CHUNK), jnp.int32),
        pltpu.VMEM((BLK, CHUNK), jnp.int32),
        pltpu.VMEM((BLK, CHUNK), jnp.float32),
        pltpu.VMEM((BLK, CHUNK), jnp.float32),
        pltpu.VMEM((BLK, CHUNK), jnp.float32),
        pltpu.VMEM_SHARED((n_pad,), jnp.float32),
        pltpu.VMEM_SHARED((n_pad,), jnp.float32),
        pltpu.VMEM_SHARED((n_pad,), jnp.float32),
        pltpu.VMEM_SHARED((n_pad,), jnp.float32),
        [pltpu.SemaphoreType.DMA] * BLK,
        pltpu.SemaphoreType.DMA,
    ]
    if combine:
        scratch += [
            pltpu.VMEM((rpt,), jnp.float32),
            pltpu.VMEM((rpt,), jnp.float32),
            pltpu.VMEM((rpt,), jnp.float32),
            pltpu.VMEM((rpt,), jnp.float32),
        ]

    def body(*refs):
        if combine:
            (q_h, d2_h, row_h, col_h, attr_h, zero_h, out_h,
             rowb, colb, attrb, m0, m1, ts0, ts1, ac0, ac1, gsem, ssem,
             qa, qb, d2b, tb) = refs
        else:
            (t0_h, t1_h, row_h, col_h, attr_h, zero_h, out_h,
             rowb, colb, attrb, m0, m1, ts0, ts1, ac0, ac1, gsem, ssem) = refs
        c = lax.axis_index("c")
        s = lax.axis_index("s")
        sl = pl.ds(s * rpt, rpt)
        pltpu.sync_copy(zero_h, ac0.at[sl])
        pltpu.sync_copy(zero_h, ac1.at[sl])
        if combine:
            pltpu.sync_copy(d2_h.at[sl], d2b)
            for p, ts in ((0, ts0), (1, ts1)):
                pltpu.sync_copy(q_h.at[0 * 2 + p, sl], qa)
                pltpu.sync_copy(q_h.at[1 * 2 + p, sl], qb)

                def stage(g, carry, qa=qa, qb=qb, tb=tb):
                    d = pl.ds(g * L, L)
                    tb[d] = (qa[d] + qb[d]) * d2b[d]
                    return carry

                lax.fori_loop(0, rpt // L, stage, 0)
                pltpu.sync_copy(tb, ts.at[sl])
        else:
            pltpu.sync_copy(t0_h.at[sl], ts0.at[sl])
            pltpu.sync_copy(t1_h.at[sl], ts1.at[sl])
        plsc.subcore_barrier()
        base = (c * NS + s) * slabs * BLK

        def slab(i, carry):
            st = base + i * BLK
            pltpu.sync_copy(row_h.at[pl.ds(st, BLK)], rowb)
            pltpu.sync_copy(col_h.at[pl.ds(st, BLK)], colb)
            pltpu.sync_copy(attr_h.at[pl.ds(st, BLK)], attrb)
            gds = []
            for j in range(BLK):
                gds.append((
                    pltpu.async_copy(ts0.at[rowb.at[j]], m0.at[j], gsem[j]),
                    pltpu.async_copy(ts1.at[rowb.at[j]], m1.at[j], gsem[j]),
                ))
            sds = []
            for j in range(BLK):
                gds[j][0].wait()
                gds[j][1].wait()
                for v in range(CHUNK // L):
                    d = pl.ds(v * L, L)
                    a = attrb[j, d]
                    m0[j, d] = m0[j, d] * a
                    m1[j, d] = m1[j, d] * a
                sds.append(pltpu.async_copy(m0.at[j], ac0.at[colb.at[j]],
                                            ssem, add=True))
                sds.append(pltpu.async_copy(m1.at[j], ac1.at[colb.at[j]],
                                            ssem, add=True))
            for d in sds:
                d.wait()
            return carry

        lax.fori_loop(0, slabs, slab, 0)
        plsc.subcore_barrier()
        pltpu.sync_copy(ac0.at[sl], out_h.at[c * 2 + 0, sl])
        pltpu.sync_copy(ac1.at[sl], out_h.at[c * 2 + 1, sl])

    return pl.kernel(
        body,
        out_type=jax.ShapeDtypeStruct((NC * 2, n_pad), jnp.float32),
        mesh=_mesh(),
        scratch_types=scratch,
    )


@functools.lru_cache(maxsize=None)
def _make_hop_rows(n_pad, nchunks, combine):
    rpt = n_pad // NS
    f = L
    slabs = nchunks // RBLK
    SR = 224
    n_sc = rpt // SR

    out_type = [jax.ShapeDtypeStruct((NC, n_pad, f), jnp.float32)]
    if combine:
        out_type.append(jax.ShapeDtypeStruct((n_pad, f), jnp.float32))

    scratch = [
        pltpu.VMEM((RBLK, CHUNK), jnp.int32),
        pltpu.VMEM((RBLK, CHUNK), jnp.int32),
        pltpu.VMEM((RBLK, CHUNK), jnp.float32),
        pltpu.VMEM((RBLK, CHUNK, f), jnp.float32),
        pltpu.VMEM_SHARED((n_pad, f), jnp.float32),
        [pltpu.SemaphoreType.DMA] * RBLK,
        pltpu.SemaphoreType.DMA,
    ]
    if combine:
        scratch += [
            pltpu.VMEM((SR, f), jnp.float32),
            pltpu.VMEM((SR, f), jnp.float32),
            pltpu.VMEM((SR,), jnp.float32),
        ]

    dnums = lax.GatherDimensionNumbers(
        offset_dims=(), collapsed_slice_dims=(0,), start_index_map=(0,))

    def body(*refs):
        if combine:
            (p_h, d2_h, row_h, col_h, attr_h, zero_h, out_h, t_h,
             rowb, colb, attrb, msgb, acc, gsem, ssem,
             pb0, pb1, d2b) = refs
        else:
            (t_h, row_h, col_h, attr_h, zero_h, out_h,
             rowb, colb, attrb, msgb, acc, gsem, ssem) = refs
        c = lax.axis_index("c")
        s = lax.axis_index("s")
        sl = pl.ds(s * rpt, rpt)
        pltpu.sync_copy(zero_h, acc.at[sl])
        if combine:
            def stage_chunk(sc, carry):
                r0 = s * rpt + sc * SR
                pltpu.sync_copy(p_h.at[0, pl.ds(r0, SR)], pb0)
                pltpu.sync_copy(p_h.at[1, pl.ds(r0, SR)], pb1)
                pltpu.sync_copy(d2_h.at[pl.ds(r0, SR)], d2b)

                def grp(g, cg):
                    av = d2b[pl.ds(g * L, L)]
                    for t in range(L):
                        nrow = g * L + t
                        bc = lax.gather(
                            av, jnp.full((L, 1), t, jnp.int32), dnums, (1,),
                            mode=lax.GatherScatterMode.PROMISE_IN_BOUNDS)
                        pb0[nrow, :] = (pb0[nrow, :] + pb1[nrow, :]) * bc
                    return cg

                lax.fori_loop(0, SR // L, grp, 0)
                pltpu.sync_copy(pb0, t_h.at[pl.ds(r0, SR)])
                return carry

            lax.fori_loop(0, n_sc, stage_chunk, 0)
        plsc.subcore_barrier()
        base = (c * NS + s) * slabs * RBLK

        def slab(i, carry):
            st = base + i * RBLK
            pltpu.sync_copy(row_h.at[pl.ds(st, RBLK)], rowb)
            pltpu.sync_copy(col_h.at[pl.ds(st, RBLK)], colb)
            pltpu.sync_copy(attr_h.at[pl.ds(st, RBLK)], attrb)
            gds = [
                pltpu.async_copy(t_h.at[rowb.at[j]], msgb.at[j], gsem[j])
                for j in range(RBLK)
            ]
            sds = []
            for j in range(RBLK):
                gds[j].wait()

                def grp(g, cg, j=j):
                    av = attrb[j, pl.ds(g * L, L)]
                    for t in range(L):
                        e = g * L + t
                        bc = lax.gather(
                            av, jnp.full((L, 1), t, jnp.int32), dnums, (1,),
                            mode=lax.GatherScatterMode.PROMISE_IN_BOUNDS)
                        msgb[j, e, :] = msgb[j, e, :] * bc
                    return cg

                lax.fori_loop(0, CHUNK // L, grp, 0)
                sds.append(pltpu.async_copy(msgb.at[j], acc.at[colb.at[j]],
                                            ssem, add=True))
            for d in sds:
                d.wait()
            return carry

        lax.fori_loop(0, slabs, slab, 0)
        plsc.subcore_barrier()
        pltpu.sync_copy(acc.at[sl], out_h.at[c, sl])

    return pl.kernel(
        body,
        out_type=tuple(out_type) if combine else out_type[0],
        mesh=_mesh(),
        compiler_params=pltpu.CompilerParams(use_tc_tiling_on_sc=False),
        scratch_types=scratch,
    )


def _planar_call(body, n_in, n_out, pr):
    spec = pl.BlockSpec((pr, 128), lambda: (0, 0))
    return pl.pallas_call(
        body,
        in_specs=[spec] * n_in,
        out_specs=[spec] * n_out if n_out > 1 else spec,
        out_shape=([jax.ShapeDtypeStruct((pr, 128), jnp.float32)] * n_out
                   if n_out > 1 else jax.ShapeDtypeStruct((pr, 128), jnp.float32)),
    )


def _tc_prep(d0, d1):
    def body(d0r, d1r, dis_o, dis2_o):
        deg = d0r[...] + d1r[...]
        dis = jnp.where(deg > 0, lax.rsqrt(jnp.maximum(deg, 1e-30)), 0.0)
        dis_o[...] = dis
        dis2_o[...] = dis * dis

    return _planar_call(body, 2, 2, d0.shape[0])(d0, d1)


def _tc_init(x0, x1, dis):
    def body(x0r, x1r, dr, t0_o, t1_o):
        d = dr[...]
        t0_o[...] = x0r[...] * d
        t1_o[...] = x1r[...] * d

    return _planar_call(body, 3, 2, x0.shape[0])(x0, x1, dis)


def _tc_bridge(hplanes, dis, w1f, bsum1, w20, b20):
    nh = len(hplanes)
    GRID = 7

    def body(*refs):
        ins = refs[:nh + 1]
        w1r, b1r, w2r, b2r = refs[nh + 1:nh + 5]
        outs = refs[nh + 5:]
        d = ins[nh][...]
        hs = [ins[0][...], ins[1][...]]
        for i in range(2, nh, 2):
            hs.append((ins[i][...] + ins[i + 1][...]) * d)
        w1 = w1r[...]
        b1 = b1r[...]
        w2 = w2r[...]
        b2 = b2r[...]
        a = []
        for i in range(16):
            z = b1[0, i]
            for j, h in enumerate(hs):
                z = z + h * w1[j, i]
            a.append(jax.nn.relu(z))
        for i in range(16):
            z = b2[0, i]
            for j in range(16):
                z = z + a[j] * w2[j, i]
            outs[i][...] = z
            outs[16 + i][...] = a[i] * d

    pr = dis.shape[0]
    spec = pl.BlockSpec((pr // GRID, 128), lambda i: (i, 0))
    return pl.pallas_call(
        body,
        grid=(GRID,),
        in_specs=[spec] * (nh + 1) + [
            pl.BlockSpec(w1f.shape, lambda i: (0, 0)),
            pl.BlockSpec(bsum1.shape, lambda i: (0, 0)),
            pl.BlockSpec(w20.shape, lambda i: (0, 0)),
            pl.BlockSpec(b20.shape, lambda i: (0, 0)),
        ],
        out_specs=[spec] * 32,
        out_shape=[jax.ShapeDtypeStruct((pr, 128), jnp.float32)] * 32,
    )(*hplanes, dis, w1f, bsum1, w20, b20)


def _tc_head(hop_planes, acc2planes, dis, w2f, wend):
    nk = len(hop_planes) // 32
    GRID = 7

    def body(*refs):
        hp = refs[0:nk * 32]
        ac = refs[nk * 32:nk * 32 + 16]
        dr = refs[nk * 32 + 16]
        w2r, wer = refs[nk * 32 + 17:nk * 32 + 19]
        o = refs[nk * 32 + 19]
        d = dr[...]
        w2 = w2r[...]
        we = wer[...]
        hs = []
        for k in range(nk):
            for j in range(16):
                hs.append((hp[k * 32 + j][...] + hp[k * 32 + 16 + j][...]) * d)
        z = None
        for i in range(16):
            zi = ac[i][...]
            for k in range(nk):
                for j in range(16):
                    zi = zi + hs[k * 16 + j] * w2[k * 16 + j, i]
            ri = jax.nn.relu(zi)
            z = ri * we[i, 0] if z is None else z + ri * we[i, 0]
        o[...] = jax.nn.sigmoid(z)

    pr = dis.shape[0]
    spec = pl.BlockSpec((pr // GRID, 128), lambda i: (i, 0))
    return pl.pallas_call(
        body,
        grid=(GRID,),
        in_specs=[spec] * (nk * 32 + 17) + [
            pl.BlockSpec(w2f.shape, lambda i: (0, 0)),
            pl.BlockSpec(wend.shape, lambda i: (0, 0)),
        ],
        out_specs=spec,
        out_shape=jax.ShapeDtypeStruct((pr, 128), jnp.float32),
    )(*hop_planes, *acc2planes, dis, w2f, wend)


def kernel(x, edge_index, edge_attr, batch, W1, b1, W2, b2, Wend):
    n = x.shape[0]
    e = edge_index.shape[1]
    kk = W1.shape[0]

    row = edge_index[0].astype(jnp.int32)
    col = edge_index[1].astype(jnp.int32)
    attr = edge_attr.astype(jnp.float32)

    epw = -(-e // (NW * CHUNK * BLK)) * CHUNK * BLK
    e_pad = NW * epw
    slabs = epw // (CHUNK * BLK)
    pad_n = e_pad - e
    pad_idx = (jnp.arange(pad_n, dtype=jnp.int32) * 37) % n
    row2 = jnp.concatenate([row, pad_idx]).reshape(-1, CHUNK)
    col2 = jnp.concatenate([col, pad_idx]).reshape(-1, CHUNK)
    attr2 = jnp.concatenate(
        [attr, jnp.zeros((pad_n,), jnp.float32)]).reshape(-1, CHUNK)

    n_pad = NS * (-(-n // (NS * CHUNK))) * CHUNK
    rpt = n_pad // NS
    pr = n_pad // 128
    x_p = jnp.zeros((n_pad, x.shape[1]), jnp.float32).at[:n].set(x)

    zero1 = jnp.zeros((rpt,), jnp.float32)
    zero16 = jnp.zeros((rpt, 16), jnp.float32)

    def pln(a):
        return a.reshape(pr, 128)

    deg_p = _make_degree(n_pad, slabs)(col2, attr2, zero1)
    dis, dis2 = _tc_prep(pln(deg_p[0]), pln(deg_p[1]))
    dis2f = dis2.reshape(n_pad)

    x0, x1 = pln(x_p[:, 0]), pln(x_p[:, 1])
    t0, t1 = _tc_init(x0, x1, dis)
    qs = []
    q = _make_hop_planar(n_pad, slabs, False)(
        t0.reshape(n_pad), t1.reshape(n_pad), row2, col2, attr2, zero1)
    qs.append(q)
    for k in range(2, kk):
        q = _make_hop_planar(n_pad, slabs, True)(
            q, dis2f, row2, col2, attr2, zero1)
        qs.append(q)

    hplanes = [x0, x1]
    for q in qs:
        for p in range(2):
            hplanes += [pln(q[0 * 2 + p]), pln(q[1 * 2 + p])]
    w1f = W1.reshape(kk * 2, 16)
    bsum1 = jnp.sum(b1, axis=0).reshape(1, 16)
    bsum2 = jnp.sum(b2, axis=0).reshape(1, 16)
    outs = _tc_bridge(hplanes, dis, w1f, bsum1, W2[0], bsum2)
    acc2planes = outs[:16]
    t2planes = outs[16:]

    t2 = jnp.stack([p.reshape(n_pad) for p in t2planes]).T
    ps = []
    p = _make_hop_rows(n_pad, slabs * BLK, False)(
        t2, row2, col2, attr2, zero16)
    ps.append(p)
    for k in range(2, kk):
        p, _ = _make_hop_rows(n_pad, slabs * BLK, True)(
            p, dis2f, row2, col2, attr2, zero16)
        ps.append(p)

    hop_planes = []
    for p in ps:
        pa = jnp.transpose(p[0])
        pb = jnp.transpose(p[1])
        hop_planes += [pln(pa[i]) for i in range(16)]
        hop_planes += [pln(pb[i]) for i in range(16)]
    w2f = W2[1:].reshape((kk - 1) * 16, 16)
    out = _tc_head(hop_planes, acc2planes, dis, w2f, Wend)
    return out.reshape(n_pad, 1)[:n]


# speedup vs baseline: 1.3190x; 1.3190x over previous
import functools

import jax
import jax.numpy as jnp
from jax import lax
from jax.experimental import pallas as pl
from jax.experimental.pallas import tpu as pltpu
from jax.experimental.pallas import tpu_sc as plsc

NC = 2
NS = 16
L = 16
NW = NC * NS
CHUNK = 128
BLK = 16
RBLK = 8
BN = 2048


def _mesh():
    return plsc.VectorSubcoreMesh(
        core_axis_name="c", subcore_axis_name="s", num_cores=NC, num_subcores=NS
    )


@functools.lru_cache(maxsize=None)
def _make_degree(n_pad, slabs):
    rpt = n_pad // NS

    @functools.partial(
        pl.kernel,
        out_type=jax.ShapeDtypeStruct((NC, n_pad), jnp.float32),
        mesh=_mesh(),
        scratch_types=[
            pltpu.VMEM((BLK, CHUNK), jnp.int32),
            pltpu.VMEM((BLK, CHUNK), jnp.float32),
            pltpu.VMEM_SHARED((n_pad,), jnp.float32),
            pltpu.SemaphoreType.DMA,
        ],
    )
    def deg_kernel(col_h, attr_h, zero_h, out_h, colb, attrb, acc, ssem):
        c = lax.axis_index("c")
        s = lax.axis_index("s")
        pltpu.sync_copy(zero_h, acc.at[pl.ds(s * rpt, rpt)])
        plsc.subcore_barrier()
        base = (c * NS + s) * slabs * BLK

        def slab(i, carry):
            st = base + i * BLK
            pltpu.sync_copy(col_h.at[pl.ds(st, BLK)], colb)
            pltpu.sync_copy(attr_h.at[pl.ds(st, BLK)], attrb)
            descs = [
                pltpu.async_copy(attrb.at[j], acc.at[colb.at[j]], ssem, add=True)
                for j in range(BLK)
            ]
            for d in descs:
                d.wait()
            return carry

        lax.fori_loop(0, slabs, slab, 0)
        plsc.subcore_barrier()
        pltpu.sync_copy(acc.at[pl.ds(s * rpt, rpt)], out_h.at[c, pl.ds(s * rpt, rpt)])

    return deg_kernel


@functools.lru_cache(maxsize=None)
def _make_hop_planar(n_pad, slabs):
    rpt = n_pad // NS

    @functools.partial(
        pl.kernel,
        out_type=jax.ShapeDtypeStruct((NC * 2, n_pad), jnp.float32),
        mesh=_mesh(),
        scratch_types=[
            pltpu.VMEM((BLK, CHUNK), jnp.int32),
            pltpu.VMEM((BLK, CHUNK), jnp.int32),
            pltpu.VMEM((BLK, CHUNK), jnp.float32),
            pltpu.VMEM((BLK, CHUNK), jnp.float32),
            pltpu.VMEM((BLK, CHUNK), jnp.float32),
            pltpu.VMEM_SHARED((n_pad,), jnp.float32),
            pltpu.VMEM_SHARED((n_pad,), jnp.float32),
            pltpu.VMEM_SHARED((n_pad,), jnp.float32),
            pltpu.VMEM_SHARED((n_pad,), jnp.float32),
            [pltpu.SemaphoreType.DMA] * BLK,
            pltpu.SemaphoreType.DMA,
        ],
    )
    def hop_kernel(t0_h, t1_h, row_h, col_h, attr_h, zero_h, out_h,
                   rowb, colb, attrb, m0, m1, ts0, ts1, ac0, ac1, gsem, ssem):
        c = lax.axis_index("c")
        s = lax.axis_index("s")
        sl = pl.ds(s * rpt, rpt)
        pltpu.sync_copy(zero_h, ac0.at[sl])
        pltpu.sync_copy(zero_h, ac1.at[sl])
        pltpu.sync_copy(t0_h.at[sl], ts0.at[sl])
        pltpu.sync_copy(t1_h.at[sl], ts1.at[sl])
        plsc.subcore_barrier()
        base = (c * NS + s) * slabs * BLK

        def slab(i, carry):
            st = base + i * BLK
            pltpu.sync_copy(row_h.at[pl.ds(st, BLK)], rowb)
            pltpu.sync_copy(col_h.at[pl.ds(st, BLK)], colb)
            pltpu.sync_copy(attr_h.at[pl.ds(st, BLK)], attrb)
            gds = []
            for j in range(BLK):
                gds.append((
                    pltpu.async_copy(ts0.at[rowb.at[j]], m0.at[j], gsem[j]),
                    pltpu.async_copy(ts1.at[rowb.at[j]], m1.at[j], gsem[j]),
                ))
            sds = []
            for j in range(BLK):
                gds[j][0].wait()
                gds[j][1].wait()
                for v in range(CHUNK // L):
                    d = pl.ds(v * L, L)
                    a = attrb[j, d]
                    m0[j, d] = m0[j, d] * a
                    m1[j, d] = m1[j, d] * a
                sds.append(pltpu.async_copy(m0.at[j], ac0.at[colb.at[j]],
                                            ssem, add=True))
                sds.append(pltpu.async_copy(m1.at[j], ac1.at[colb.at[j]],
                                            ssem, add=True))
            for d in sds:
                d.wait()
            return carry

        lax.fori_loop(0, slabs, slab, 0)
        plsc.subcore_barrier()
        pltpu.sync_copy(ac0.at[sl], out_h.at[c * 2 + 0, sl])
        pltpu.sync_copy(ac1.at[sl], out_h.at[c * 2 + 1, sl])

    return hop_kernel


@functools.lru_cache(maxsize=None)
def _make_hop_rows(n_pad, nchunks):
    rpt = n_pad // NS
    f = L
    slabs = nchunks // RBLK

    @functools.partial(
        pl.kernel,
        out_type=jax.ShapeDtypeStruct((NC, n_pad, f), jnp.float32),
        mesh=_mesh(),
        compiler_params=pltpu.CompilerParams(use_tc_tiling_on_sc=False),
        scratch_types=[
            pltpu.VMEM((RBLK, CHUNK), jnp.int32),
            pltpu.VMEM((RBLK, CHUNK), jnp.int32),
            pltpu.VMEM((RBLK, CHUNK), jnp.float32),
            pltpu.VMEM((RBLK, CHUNK, f), jnp.float32),
            pltpu.VMEM_SHARED((n_pad, f), jnp.float32),
            [pltpu.SemaphoreType.DMA] * RBLK,
            pltpu.SemaphoreType.DMA,
        ],
    )
    def hop_kernel(t_h, row_h, col_h, attr_h, zero_h, out_h,
                   rowb, colb, attrb, msgb, acc, gsem, ssem):
        c = lax.axis_index("c")
        s = lax.axis_index("s")
        sl = pl.ds(s * rpt, rpt)
        pltpu.sync_copy(zero_h, acc.at[sl])
        plsc.subcore_barrier()
        base = (c * NS + s) * slabs * RBLK
        dnums = lax.GatherDimensionNumbers(
            offset_dims=(), collapsed_slice_dims=(0,), start_index_map=(0,))

        def slab(i, carry):
            st = base + i * RBLK
            pltpu.sync_copy(row_h.at[pl.ds(st, RBLK)], rowb)
            pltpu.sync_copy(col_h.at[pl.ds(st, RBLK)], colb)
            pltpu.sync_copy(attr_h.at[pl.ds(st, RBLK)], attrb)
            gds = [
                pltpu.async_copy(t_h.at[rowb.at[j]], msgb.at[j], gsem[j])
                for j in range(RBLK)
            ]
            sds = []
            for j in range(RBLK):
                gds[j].wait()

                def grp(g, cg, j=j):
                    av = attrb[j, pl.ds(g * L, L)]
                    for t in range(L):
                        e = g * L + t
                        bc = lax.gather(
                            av, jnp.full((L, 1), t, jnp.int32), dnums, (1,),
                            mode=lax.GatherScatterMode.PROMISE_IN_BOUNDS)
                        msgb[j, e, :] = msgb[j, e, :] * bc
                    return cg

                lax.fori_loop(0, CHUNK // L, grp, 0)
                sds.append(pltpu.async_copy(msgb.at[j], acc.at[colb.at[j]],
                                            ssem, add=True))
            for d in sds:
                d.wait()
            return carry

        lax.fori_loop(0, slabs, slab, 0)
        plsc.subcore_barrier()
        pltpu.sync_copy(acc.at[sl], out_h.at[c, sl])

    return hop_kernel


def _row_spec(bf):
    return pl.BlockSpec((BN, bf), lambda i: (i, 0))


def _full_spec(shape):
    nd = len(shape)
    return pl.BlockSpec(shape, lambda i: (0,) * nd)


def _tc_prep(d0, d1):
    def body(d0r, d1r, o):
        deg = d0r[...] + d1r[...]
        o[...] = jnp.where(deg > 0, lax.rsqrt(jnp.maximum(deg, 1e-30)), 0.0)

    n_pad = d0.shape[0]
    return pl.pallas_call(
        body,
        grid=(n_pad // BN,),
        in_specs=[_row_spec(1), _row_spec(1)],
        out_specs=_row_spec(1),
        out_shape=jax.ShapeDtypeStruct((n_pad, 1), jnp.float32),
    )(d0, d1)


def _tc_init(x0, x1, dis, w, b):
    def body(x0r, x1r, dr, wr, br, acc_o, t0_o, t1_o):
        acc_o[...] = x0r[...] * wr[0:1, :] + x1r[...] * wr[1:2, :] + br[...]
        t0_o[...] = x0r[...] * dr[...]
        t1_o[...] = x1r[...] * dr[...]

    n_pad = x0.shape[0]
    o1 = jax.ShapeDtypeStruct((n_pad, 1), jnp.float32)
    return pl.pallas_call(
        body,
        grid=(n_pad // BN,),
        in_specs=[_row_spec(1), _row_spec(1), _row_spec(1),
                  _full_spec(w.shape), _full_spec(b.shape)],
        out_specs=[_row_spec(16), _row_spec(1), _row_spec(1)],
        out_shape=[jax.ShapeDtypeStruct((n_pad, 16), jnp.float32), o1, o1],
    )(x0, x1, dis, w, b)


def _tc_combine2(q00, q01, q10, q11, dis, acc, w, b):
    def body(a0, a1, b0, b1, dr, ar, wr, br, acc_o, t0_o, t1_o):
        d = dr[...]
        h0 = (a0[...] + b0[...]) * d
        h1 = (a1[...] + b1[...]) * d
        acc_o[...] = ar[...] + h0 * wr[0:1, :] + h1 * wr[1:2, :] + br[...]
        t0_o[...] = h0 * d
        t1_o[...] = h1 * d

    n_pad = q00.shape[0]
    o1 = jax.ShapeDtypeStruct((n_pad, 1), jnp.float32)
    return pl.pallas_call(
        body,
        grid=(n_pad // BN,),
        in_specs=[_row_spec(1)] * 5 + [_row_spec(16),
                  _full_spec(w.shape), _full_spec(b.shape)],
        out_specs=[_row_spec(16), _row_spec(1), _row_spec(1)],
        out_shape=[jax.ShapeDtypeStruct((n_pad, 16), jnp.float32), o1, o1],
    )(q00, q01, q10, q11, dis, acc, w, b)


def _tc_bridge(q00, q01, q10, q11, dis, acc, w14, b14, w20, b20):
    def body(a0, a1, b0, b1, dr, ar, w1r, b1r, w2r, b2r, acc_o, t_o):
        d = dr[...]
        h0 = (a0[...] + b0[...]) * d
        h1 = (a1[...] + b1[...]) * d
        a = jax.nn.relu(ar[...] + h0 * w1r[0:1, :] + h1 * w1r[1:2, :] + b1r[...])
        acc_o[...] = (
            jnp.dot(a, w2r[...], preferred_element_type=jnp.float32) + b2r[...]
        )
        t_o[...] = a * d

    n_pad = q00.shape[0]
    return pl.pallas_call(
        body,
        grid=(n_pad // BN,),
        in_specs=[_row_spec(1)] * 5 + [_row_spec(16),
                  _full_spec(w14.shape), _full_spec(b14.shape),
                  _full_spec(w20.shape), _full_spec(b20.shape)],
        out_specs=[_row_spec(16), _row_spec(16)],
        out_shape=[jax.ShapeDtypeStruct((n_pad, 16), jnp.float32),
                   jax.ShapeDtypeStruct((n_pad, 16), jnp.float32)],
    )(q00, q01, q10, q11, dis, acc, w14, b14, w20, b20)


def _tc_combine16(p0, p1, dis, acc, w, b):
    def body(p0r, p1r, dr, ar, wr, br, acc_o, t_o):
        d = dr[...]
        h = (p0r[...] + p1r[...]) * d
        acc_o[...] = (
            ar[...] + jnp.dot(h, wr[...], preferred_element_type=jnp.float32)
            + br[...]
        )
        t_o[...] = h * d

    n_pad = p0.shape[0]
    return pl.pallas_call(
        body,
        grid=(n_pad // BN,),
        in_specs=[_row_spec(16), _row_spec(16), _row_spec(1), _row_spec(16),
                  _full_spec(w.shape), _full_spec(b.shape)],
        out_specs=[_row_spec(16), _row_spec(16)],
        out_shape=[jax.ShapeDtypeStruct((n_pad, 16), jnp.float32),
                   jax.ShapeDtypeStruct((n_pad, 16), jnp.float32)],
    )(p0, p1, dis, acc, w, b)


def _tc_final(p0, p1, dis, acc, w24, b24, wend):
    def body(p0r, p1r, dr, ar, wr, br, wer, o):
        h = (p0r[...] + p1r[...]) * dr[...]
        a = jax.nn.relu(
            ar[...] + jnp.dot(h, wr[...], preferred_element_type=jnp.float32)
            + br[...]
        )
        o[...] = jax.nn.sigmoid(
            jnp.dot(a, wer[...], preferred_element_type=jnp.float32)
        )

    n_pad = p0.shape[0]
    return pl.pallas_call(
        body,
        grid=(n_pad // BN,),
        in_specs=[_row_spec(16), _row_spec(16), _row_spec(1), _row_spec(16),
                  _full_spec(w24.shape), _full_spec(b24.shape),
                  _full_spec(wend.shape)],
        out_specs=_row_spec(1),
        out_shape=jax.ShapeDtypeStruct((n_pad, 1), jnp.float32),
    )(p0, p1, dis, acc, w24, b24, wend)


def kernel(x, edge_index, edge_attr, batch, W1, b1, W2, b2, Wend):
    n = x.shape[0]
    e = edge_index.shape[1]
    kk = W1.shape[0]

    row = edge_index[0].astype(jnp.int32)
    col = edge_index[1].astype(jnp.int32)
    attr = edge_attr.astype(jnp.float32)

    epw = -(-e // (NW * CHUNK * BLK)) * CHUNK * BLK
    e_pad = NW * epw
    slabs = epw // (CHUNK * BLK)
    pad_n = e_pad - e
    pad_idx = (jnp.arange(pad_n, dtype=jnp.int32) * 37) % n
    row2 = jnp.concatenate([row, pad_idx]).reshape(-1, CHUNK)
    col2 = jnp.concatenate([col, pad_idx]).reshape(-1, CHUNK)
    attr2 = jnp.concatenate(
        [attr, jnp.zeros((pad_n,), jnp.float32)]).reshape(-1, CHUNK)

    n_pad = NS * (-(-n // (NS * CHUNK))) * CHUNK
    rpt = n_pad // NS
    x_p = jnp.zeros((n_pad, x.shape[1]), jnp.float32).at[:n].set(x)

    zero1 = jnp.zeros((rpt,), jnp.float32)
    zero16 = jnp.zeros((rpt, 16), jnp.float32)

    deg_p = _make_degree(n_pad, slabs)(col2, attr2, zero1)
    dis = _tc_prep(deg_p[0].reshape(n_pad, 1), deg_p[1].reshape(n_pad, 1))

    acc, t0, t1 = _tc_init(x_p[:, 0:1], x_p[:, 1:2], dis,
                           W1[0], b1[0].reshape(1, -1))
    hop2 = _make_hop_planar(n_pad, slabs)
    for k in range(1, kk):
        q = hop2(t0.reshape(n_pad), t1.reshape(n_pad),
                 row2, col2, attr2, zero1)
        planes = [q[i].reshape(n_pad, 1) for i in range(4)]
        if k < kk - 1:
            acc, t0, t1 = _tc_combine2(*planes, dis, acc,
                                       W1[k], b1[k].reshape(1, -1))
        else:
            acc, t = _tc_bridge(*planes, dis, acc,
                                W1[k], b1[k].reshape(1, -1),
                                W2[0], b2[0].reshape(1, -1))

    hop16 = _make_hop_rows(n_pad, slabs * BLK)
    for k in range(1, kk):
        p = hop16(t, row2, col2, attr2, zero16)
        if k < kk - 1:
            acc, t = _tc_combine16(p[0], p[1], dis, acc,
                                   W2[k], b2[k].reshape(1, -1))
        else:
            out = _tc_final(p[0], p[1], dis, acc,
                            W2[k], b2[k].reshape(1, -1), Wend)

    return out[:n]
